# Initial kernel scaffold; baseline (speedup 1.0000x reference)
import jax
import jax.numpy as jnp
from jax.experimental import pallas as pl


def kernel(q, e1, e2):
    raise NotImplementedError("write your pallas kernel here")


# speedup vs baseline: 5.4913x; 5.4913x over previous
import jax
import jax.numpy as jnp
from jax.experimental import pallas as pl
from jax.experimental.pallas import tpu as pltpu

H = 8
S = 256
T = 1024
D = 64
SZ_B = 16
B = SZ_B * H
RATIO = T // S
W = 2 * S

T_BLK = 256


def _rel_attn_kernel(q_ref, c_ref, o_ref):
    j = pl.program_id(2)
    sc = jax.lax.dot_general(
        q_ref[0], c_ref[0],
        (((1,), (1,)), ((), ())),
        preferred_element_type=jnp.float32,
    )
    r = jax.lax.broadcasted_iota(jnp.int32, (T_BLK, 1), 0)
    shift = (S - 1) - (j * T_BLK + r) // RATIO
    x = sc
    for k in range(8):
        rolled = jnp.roll(x, -(1 << k), axis=1)
        x = jnp.where(((shift >> k) & 1) == 1, rolled, x)
    o_ref[0] = x[:, :S]


@jax.jit
def kernel(q, e1, e2):
    e1h = e1.reshape(H, S, D)
    e2h = e2.reshape(H, S, D)
    c = jnp.concatenate(
        [e1h, e2h[:, 1:, :], jnp.zeros((H, 1, D), e2h.dtype)], axis=1)

    grid = (H, SZ_B, T // T_BLK)
    return pl.pallas_call(
        _rel_attn_kernel,
        grid=grid,
        in_specs=[
            pl.BlockSpec((1, T_BLK, D), lambda h, b, j: (b * H + h, j, 0)),
            pl.BlockSpec((1, W, D), lambda h, b, j: (h, 0, 0)),
        ],
        out_specs=pl.BlockSpec((1, T_BLK, S), lambda h, b, j: (b * H + h, j, 0)),
        out_shape=jax.ShapeDtypeStruct((B, T, S), jnp.float32),
        compiler_params=pltpu.CompilerParams(
            dimension_semantics=("parallel", "parallel", "arbitrary"),
        ),
    )(q, c)
